# baseline (device time: 70397 ns/iter reference)
import jax
import jax.numpy as jnp
from jax import lax
from jax.experimental import pallas as pl
from jax.experimental.pallas import tpu as pltpu

N_DEV = 4
S = 1024
D = 2048
DC = 128
H = 16
HP = H // N_DEV
DH = 128
DR = 32
HD = HP * DH
SCALE = float((DH + DR) ** -0.5)
LOG2E = 1.4426950408889634
QSCALE = SCALE * LOG2E
BF = jnp.bfloat16
F32 = jnp.float32


def _mla_body(x_ref, wdkv_ref, wuk_ref, wuv_ref, wkr_ref, wq_ref, wqr_ref,
              o_ref,
              cbuf, wkb, wvb, wkbuf, wvbuf, wqf, obuf,
              gsend, grecv, bsend, brecv, fsend, frecv, loc_a, loc_b):
    my = lax.axis_index("i")
    right = lax.rem(my + 1, N_DEV)
    left = lax.rem(my + 3, N_DEV)

    wq_dma = pltpu.make_async_copy(
        wq_ref.at[:, pl.ds(my * HD, HD)], wqf, loc_a.at[2])
    wq_dma.start()

    xb = x_ref[0].astype(BF)
    kr = jnp.dot(xb, wkr_ref[...].astype(BF),
                 preferred_element_type=F32).astype(BF)
    cbuf[0] = jnp.dot(xb, wdkv_ref[...].astype(BF),
                      preferred_element_type=F32).astype(BF)
    wkb[...] = wuk_ref[...].astype(BF)
    wvb[...] = wuv_ref[...].astype(BF)

    barrier = pltpu.get_barrier_semaphore()
    for k in range(1, N_DEV):
        pl.semaphore_signal(barrier, inc=1,
                            device_id=(lax.rem(my + k, N_DEV),),
                            device_id_type=pl.DeviceIdType.MESH)
    pl.semaphore_wait(barrier, N_DEV - 1)

    own_k = pltpu.make_async_copy(
        wkb.at[:, pl.ds(my * HD, HD)], wkbuf.at[0], loc_a.at[0])
    own_v = pltpu.make_async_copy(
        wvb.at[:, pl.ds(my * HD, HD)], wvbuf.at[0], loc_a.at[1])
    own_k.start()
    own_v.start()

    def _gather_rdmas(k, start):
        d = lax.rem(my + k, N_DEV)
        slot = N_DEV - k
        rdmas = []
        for t, (src, dst) in enumerate((
            (cbuf.at[0], cbuf.at[slot]),
            (wkb.at[:, pl.ds(d * HD, HD)], wkbuf.at[slot]),
            (wvb.at[:, pl.ds(d * HD, HD)], wvbuf.at[slot]),
        )):
            rdma = pltpu.make_async_remote_copy(
                src_ref=src, dst_ref=dst,
                send_sem=gsend.at[slot - 1, t],
                recv_sem=grecv.at[slot - 1, t],
                device_id=(d,), device_id_type=pl.DeviceIdType.MESH,
            )
            if start:
                rdma.start()
            rdmas.append(rdma)
        return rdmas

    for k in range(1, N_DEV):
        _gather_rdmas(k, start=True)

    wq_dma.wait()
    wqb = (wqf[...] * QSCALE).astype(BF)
    q = jnp.dot(xb, wqb, preferred_element_type=F32).astype(BF)
    qrs = [jnp.dot(xb, wqr_ref[h], preferred_element_type=F32).astype(BF)
           for h in range(HP)]
    own_k.wait()
    own_v.wait()
    k_acc = jnp.dot(cbuf[0], wkbuf[0], preferred_element_type=F32)
    v_acc = jnp.dot(cbuf[0], wvbuf[0], preferred_element_type=F32)
    for slot in (3, 2, 1):
        for rdma in _gather_rdmas(N_DEV - slot, start=False):
            rdma.wait()
        k_acc += jnp.dot(cbuf[slot], wkbuf[slot], preferred_element_type=F32)
        v_acc += jnp.dot(cbuf[slot], wvbuf[slot], preferred_element_type=F32)
    kb = k_acc.astype(BF)
    vb = v_acc.astype(BF)
    ones = jnp.ones((S, DH), BF)


    def _direct(hh, start):
        col = (my * HP + hh) * DH
        rdmas = [pltpu.make_async_copy(
            obuf.at[hh], o_ref.at[:, pl.ds(col, DH)], loc_b.at[hh])]
        for idx, d in ((1, right), (0, left)):
            rdmas.append(pltpu.make_async_remote_copy(
                src_ref=obuf.at[hh],
                dst_ref=o_ref.at[:, pl.ds(col, DH)],
                send_sem=bsend.at[idx, hh],
                recv_sem=brecv.at[idx, hh],
                device_id=(d,), device_id_type=pl.DeviceIdType.MESH,
            ))
        if start:
            for rdma in rdmas:
                rdma.start()
        return rdmas

    def _forward(origin, hh, dest, fidx, fslot, start):
        col = (origin * HP + hh) * DH
        rdma = pltpu.make_async_remote_copy(
            src_ref=o_ref.at[:, pl.ds(col, DH)],
            dst_ref=o_ref.at[:, pl.ds(col, DH)],
            send_sem=fsend.at[fidx, fslot],
            recv_sem=frecv.at[fidx, fslot],
            device_id=(dest,), device_id_type=pl.DeviceIdType.MESH,
        )
        if start:
            rdma.start()
        return rdma

    for h in range(HP):
        qcat = jnp.concatenate([q[:, h * DH:(h + 1) * DH], qrs[h]], axis=1)
        kcat = jnp.concatenate([kb[:, h * DH:(h + 1) * DH], kr], axis=1)
        vd = jnp.concatenate([vb[:, h * DH:(h + 1) * DH], ones], axis=1)
        s = lax.dot_general(qcat, kcat, (((1,), (1,)), ((), ())),
                            preferred_element_type=F32)
        p = jnp.exp2(s.astype(BF))
        r = jnp.dot(p, vd, preferred_element_type=F32)
        denom = r[:, DH:DH + 1]
        obuf[h] = (r[:, :DH] * (1.0 / denom)).astype(BF)
        _direct(h, start=True)

    for h in (0, 1):
        _direct(h, start=False)[2].wait_recv()
        _forward(left, h, right, 1, h, start=True)
    for h in (2, 3):
        _direct(h, start=False)[1].wait_recv()
        _forward(right, h, left, 0, h - 2, start=True)

    for h in range(HP):
        rd = _direct(h, start=False)
        rd[0].wait()
        rd[1].wait_send()
        rd[2].wait_send()
    for h in (2, 3):
        _direct(h, start=False)[2].wait_recv()
    for h in (0, 1):
        _direct(h, start=False)[1].wait_recv()
    for h in (0, 1):
        fw = _forward(left, h, right, 1, h, start=False)
        fw.wait_send()
        fw.wait_recv()
    for h in (2, 3):
        fw = _forward(right, h, left, 0, h - 2, start=False)
        fw.wait_send()
        fw.wait_recv()


def _proj_body(o_ref, wo_ref, out_ref):
    out_ref[...] = jnp.dot(o_ref[...], wo_ref[...].astype(BF),
                           preferred_element_type=F32)


def kernel(x, Wdkv, Wuk, Wuv, Wq, Wqr, Wkr, Wo):
    my = lax.axis_index("i")

    wqr3 = jnp.transpose((Wqr * QSCALE).astype(BF).reshape(D, H, DR),
                         (1, 0, 2))
    wqr_own = lax.dynamic_slice(wqr3, (my * HP, 0, 0), (HP, D, DR))

    O = pl.pallas_call(
        _mla_body,
        out_shape=jax.ShapeDtypeStruct((S, D), BF),
        in_specs=[pl.BlockSpec(memory_space=pltpu.VMEM)] * 5
        + [pl.BlockSpec(memory_space=pltpu.MemorySpace.HBM),
           pl.BlockSpec(memory_space=pltpu.VMEM)],
        out_specs=pl.BlockSpec(memory_space=pltpu.VMEM),
        scratch_shapes=[
            pltpu.VMEM((N_DEV, S, DC), BF),
            pltpu.VMEM((DC, D), BF),
            pltpu.VMEM((DC, D), BF),
            pltpu.VMEM((N_DEV, DC, HD), BF),
            pltpu.VMEM((N_DEV, DC, HD), BF),
            pltpu.VMEM((D, HD), F32),
            pltpu.VMEM((HP, S, DH), BF),
            pltpu.SemaphoreType.DMA((N_DEV - 1, 3)),
            pltpu.SemaphoreType.DMA((N_DEV - 1, 3)),
            pltpu.SemaphoreType.DMA((2, HP)),
            pltpu.SemaphoreType.DMA((2, HP)),
            pltpu.SemaphoreType.DMA((2, 2)),
            pltpu.SemaphoreType.DMA((2, 2)),
            pltpu.SemaphoreType.DMA((3,)),
            pltpu.SemaphoreType.DMA((HP,)),
        ],
        compiler_params=pltpu.CompilerParams(collective_id=0),
    )(x, Wdkv, Wuk, Wuv, Wkr, Wq, wqr_own)

    out = pl.pallas_call(
        _proj_body,
        grid=(4,),
        out_shape=jax.ShapeDtypeStruct((S, D), F32),
        in_specs=[
            pl.BlockSpec((S, D), lambda j: (0, 0)),
            pl.BlockSpec((D, D // 4), lambda j: (0, j)),
        ],
        out_specs=pl.BlockSpec((S, D // 4), lambda j: (0, j)),
    )(O, Wo)

    return out[None]


# device time: 68295 ns/iter; 1.0308x vs baseline; 1.0308x over previous
import jax
import jax.numpy as jnp
from jax import lax
from jax.experimental import pallas as pl
from jax.experimental.pallas import tpu as pltpu

N_DEV = 4
S = 1024
D = 2048
DC = 128
H = 16
HP = H // N_DEV
DH = 128
DR = 32
DQ = DH + DR
HD = HP * DH
SCALE = float((DH + DR) ** -0.5)
LOG2E = 1.4426950408889634
QSCALE = SCALE * LOG2E
BF = jnp.bfloat16
F32 = jnp.float32


def _gather_body(x_ref, wdkv_ref, wuk_ref, wuv_ref, wkr_ref,
                 wq_ref, wqr_ref,
                 qc_ref, kc_ref, vd_ref,
                 cbuf, wkb, wvb, wkbuf, wvbuf, wqf,
                 send_sems, recv_sems, loc_sems):
    my = lax.axis_index("i")

    wq_dma = pltpu.make_async_copy(
        wq_ref.at[:, pl.ds(my * HD, HD)], wqf, loc_sems.at[2])
    wq_dma.start()

    xb = x_ref[0].astype(BF)
    kr = jnp.dot(xb, wkr_ref[...].astype(BF),
                 preferred_element_type=F32).astype(BF)
    cbuf[0] = jnp.dot(xb, wdkv_ref[...].astype(BF),
                      preferred_element_type=F32).astype(BF)
    wkb[...] = wuk_ref[...].astype(BF)
    wvb[...] = wuv_ref[...].astype(BF)

    barrier = pltpu.get_barrier_semaphore()
    for k in range(1, N_DEV):
        pl.semaphore_signal(barrier, inc=1,
                            device_id=(lax.rem(my + k, N_DEV),),
                            device_id_type=pl.DeviceIdType.MESH)
    pl.semaphore_wait(barrier, N_DEV - 1)

    own_k = pltpu.make_async_copy(
        wkb.at[:, pl.ds(my * HD, HD)], wkbuf.at[0], loc_sems.at[0])
    own_v = pltpu.make_async_copy(
        wvb.at[:, pl.ds(my * HD, HD)], wvbuf.at[0], loc_sems.at[1])
    own_k.start()
    own_v.start()

    def _rdmas(k, start):
        d = lax.rem(my + k, N_DEV)
        slot = N_DEV - k
        rdmas = []
        for t, (src, dst) in enumerate((
            (cbuf.at[0], cbuf.at[slot]),
            (wkb.at[:, pl.ds(d * HD, HD)], wkbuf.at[slot]),
            (wvb.at[:, pl.ds(d * HD, HD)], wvbuf.at[slot]),
        )):
            rdma = pltpu.make_async_remote_copy(
                src_ref=src, dst_ref=dst,
                send_sem=send_sems.at[slot - 1, t],
                recv_sem=recv_sems.at[slot - 1, t],
                device_id=(d,), device_id_type=pl.DeviceIdType.MESH,
            )
            if start:
                rdma.start()
            rdmas.append(rdma)
        return rdmas

    for k in range(1, N_DEV):
        _rdmas(k, start=True)

    wq_dma.wait()
    wqb = (wqf[...] * QSCALE).astype(BF)
    q = jnp.dot(xb, wqb, preferred_element_type=F32).astype(BF)
    for h in range(HP):
        qr_h = jnp.dot(xb, wqr_ref[h],
                       preferred_element_type=F32).astype(BF)
        qc_ref[h] = jnp.concatenate(
            [q[:, h * DH:(h + 1) * DH], qr_h], axis=1)
    own_k.wait()
    own_v.wait()
    k_acc = jnp.dot(cbuf[0], wkbuf[0], preferred_element_type=F32)
    v_acc = jnp.dot(cbuf[0], wvbuf[0], preferred_element_type=F32)
    for slot in (3, 2, 1):
        for rdma in _rdmas(N_DEV - slot, start=False):
            rdma.wait()
        k_acc += jnp.dot(cbuf[slot], wkbuf[slot], preferred_element_type=F32)
        v_acc += jnp.dot(cbuf[slot], wvbuf[slot], preferred_element_type=F32)
    kb = k_acc.astype(BF)
    vb = v_acc.astype(BF)
    ones = jnp.ones((S, DH), BF)
    for h in range(HP):
        kc_ref[h] = jnp.concatenate(
            [kb[:, h * DH:(h + 1) * DH], kr], axis=1)
        vd_ref[h] = jnp.concatenate(
            [vb[:, h * DH:(h + 1) * DH], ones], axis=1)


def _attn_body(qc_ref, kc_ref, vd_ref,
               o_ref, obuf, send_sems, recv_sems,
               fsend_sems, frecv_sems, loc_sems):
    my = lax.axis_index("i")
    right = lax.rem(my + 1, N_DEV)
    left = lax.rem(my + 3, N_DEV)

    barrier = pltpu.get_barrier_semaphore()
    for nbr in (left, right):
        pl.semaphore_signal(barrier, inc=1, device_id=(nbr,),
                            device_id_type=pl.DeviceIdType.MESH)
    pl.semaphore_wait(barrier, 2)

    def _direct(hh, start):
        col = (my * HP + hh) * DH
        rdmas = [pltpu.make_async_copy(
            obuf.at[hh], o_ref.at[:, pl.ds(col, DH)], loc_sems.at[hh])]
        for idx, d in ((1, right), (0, left)):
            rdmas.append(pltpu.make_async_remote_copy(
                src_ref=obuf.at[hh],
                dst_ref=o_ref.at[:, pl.ds(col, DH)],
                send_sem=send_sems.at[idx, hh],
                recv_sem=recv_sems.at[idx, hh],
                device_id=(d,), device_id_type=pl.DeviceIdType.MESH,
            ))
        if start:
            for rdma in rdmas:
                rdma.start()
        return rdmas

    def _forward(origin, hh, dest, fidx, fslot, start):
        col = (origin * HP + hh) * DH
        rdma = pltpu.make_async_remote_copy(
            src_ref=o_ref.at[:, pl.ds(col, DH)],
            dst_ref=o_ref.at[:, pl.ds(col, DH)],
            send_sem=fsend_sems.at[fidx, fslot],
            recv_sem=frecv_sems.at[fidx, fslot],
            device_id=(dest,), device_id_type=pl.DeviceIdType.MESH,
        )
        if start:
            rdma.start()
        return rdma

    for h in range(HP):
        s = lax.dot_general(qc_ref[h], kc_ref[h], (((1,), (1,)), ((), ())),
                            preferred_element_type=F32)
        p = jnp.exp2(s.astype(BF))
        r = jnp.dot(p, vd_ref[h], preferred_element_type=F32)
        denom = r[:, DH:DH + 1]
        obuf[h] = (r[:, :DH] * (1.0 / denom)).astype(BF)
        _direct(h, start=True)

    for h in (0, 1):
        _direct(h, start=False)[2].wait_recv()
        _forward(left, h, right, 1, h, start=True)
    for h in (2, 3):
        _direct(h, start=False)[1].wait_recv()
        _forward(right, h, left, 0, h - 2, start=True)

    for h in range(HP):
        rd = _direct(h, start=False)
        rd[0].wait()
        rd[1].wait_send()
        rd[2].wait_send()
    for h in (2, 3):
        _direct(h, start=False)[2].wait_recv()
    for h in (0, 1):
        _direct(h, start=False)[1].wait_recv()
    for h in (0, 1):
        fw = _forward(left, h, right, 1, h, start=False)
        fw.wait_send()
        fw.wait_recv()
    for h in (2, 3):
        fw = _forward(right, h, left, 0, h - 2, start=False)
        fw.wait_send()
        fw.wait_recv()


def _proj_body(o_ref, wo_ref, out_ref):
    out_ref[...] = jnp.dot(o_ref[...], wo_ref[...].astype(BF),
                           preferred_element_type=F32)


def kernel(x, Wdkv, Wuk, Wuv, Wq, Wqr, Wkr, Wo):
    my = lax.axis_index("i")

    wqr3 = jnp.transpose((Wqr * QSCALE).astype(BF).reshape(D, H, DR),
                         (1, 0, 2))
    wqr_own = lax.dynamic_slice(wqr3, (my * HP, 0, 0), (HP, D, DR))

    QC, KC, V = pl.pallas_call(
        _gather_body,
        out_shape=[
            jax.ShapeDtypeStruct((HP, S, DQ), BF),
            jax.ShapeDtypeStruct((HP, S, DQ), BF),
            jax.ShapeDtypeStruct((HP, S, 2 * DH), BF),
        ],
        in_specs=[pl.BlockSpec(memory_space=pltpu.VMEM)] * 5
        + [pl.BlockSpec(memory_space=pltpu.MemorySpace.HBM),
           pl.BlockSpec(memory_space=pltpu.VMEM)],
        out_specs=[pl.BlockSpec(memory_space=pltpu.VMEM)] * 3,
        scratch_shapes=[
            pltpu.VMEM((N_DEV, S, DC), BF),
            pltpu.VMEM((DC, D), BF),
            pltpu.VMEM((DC, D), BF),
            pltpu.VMEM((N_DEV, DC, HD), BF),
            pltpu.VMEM((N_DEV, DC, HD), BF),
            pltpu.VMEM((D, HD), F32),
            pltpu.SemaphoreType.DMA((N_DEV - 1, 3)),
            pltpu.SemaphoreType.DMA((N_DEV - 1, 3)),
            pltpu.SemaphoreType.DMA((3,)),
        ],
        compiler_params=pltpu.CompilerParams(collective_id=0),
    )(x, Wdkv, Wuk, Wuv, Wkr, Wq, wqr_own)

    O = pl.pallas_call(
        _attn_body,
        out_shape=jax.ShapeDtypeStruct((S, D), BF),
        in_specs=[pl.BlockSpec(memory_space=pltpu.VMEM)] * 3,
        out_specs=pl.BlockSpec(memory_space=pltpu.VMEM),
        scratch_shapes=[
            pltpu.VMEM((HP, S, DH), BF),
            pltpu.SemaphoreType.DMA((2, HP)),
            pltpu.SemaphoreType.DMA((2, HP)),
            pltpu.SemaphoreType.DMA((2, 2)),
            pltpu.SemaphoreType.DMA((2, 2)),
            pltpu.SemaphoreType.DMA((HP,)),
        ],
        compiler_params=pltpu.CompilerParams(collective_id=1),
    )(QC, KC, V)

    out = pl.pallas_call(
        _proj_body,
        grid=(4,),
        out_shape=jax.ShapeDtypeStruct((S, D), F32),
        in_specs=[
            pl.BlockSpec((S, D), lambda j: (0, 0)),
            pl.BlockSpec((D, D // 4), lambda j: (0, j)),
        ],
        out_specs=pl.BlockSpec((S, D // 4), lambda j: (0, j)),
    )(O, Wo)

    return out[None]
